# baseline (speedup 1.0000x reference)
import functools

import jax
import jax.numpy as jnp
from jax import lax
from jax.experimental import pallas as pl
from jax.experimental.pallas import tpu as pltpu
from jax.experimental.pallas import tpu_sc as plsc

N = 10000
IN_DIM = 256
HID = 512
EPS = 1e-5
NC, NS = 2, 16
NW = NC * NS
E_PAD = 163840
EPW = E_PAD // NW
NB = EPW // 128
ACC = 10112
RPW = ACC // NS
RB = 400
NRB = N // RB


def _sc_mesh():
    return plsc.VectorSubcoreMesh(core_axis_name="c", subcore_axis_name="s",
                                  num_cores=NC, num_subcores=NS)


def _sc_degree(col, zeros_c, ones_b):
    @functools.partial(
        pl.kernel,
        out_type=jax.ShapeDtypeStruct((NC, ACC, 128), jnp.float32),
        mesh=_sc_mesh(),
        scratch_types=[
            pltpu.VMEM((128,), jnp.int32),
            pltpu.VMEM((128, 128), jnp.float32),
            pltpu.VMEM_SHARED((ACC, 128), jnp.float32),
        ],
    )
    def k(col_h, zeros_h, ones_h, out_h, idx_v, ones_v, acc_sh):
        c = lax.axis_index("c")
        s = lax.axis_index("s")
        wb = (c * NS + s) * NB
        pltpu.sync_copy(ones_h, ones_v)
        pltpu.sync_copy(zeros_h, acc_sh.at[pl.ds(s * RPW, RPW)])
        plsc.subcore_barrier()

        @pl.loop(0, NB)
        def _(b):
            pltpu.sync_copy(col_h.at[wb + b], idx_v)
            pltpu.sync_copy(ones_v, acc_sh.at[idx_v], add=True)

        plsc.subcore_barrier()
        pltpu.sync_copy(acc_sh.at[pl.ds(s * RPW, RPW)],
                        out_h.at[c, pl.ds(s * RPW, RPW)])

    return k(col, zeros_c, ones_b)


def _sc_aggregate(hs, row, col, zeros_c):
    P, _, C = hs.shape

    @functools.partial(
        pl.kernel,
        out_type=jax.ShapeDtypeStruct((NC, P, ACC, C), jnp.float32),
        mesh=_sc_mesh(),
        scratch_types=[
            pltpu.VMEM((NB, 128), jnp.int32),
            pltpu.VMEM((NB, 128), jnp.int32),
            pltpu.VMEM((2, 128, C), jnp.float32),
            pltpu.VMEM_SHARED((ACC, C), jnp.float32),
            pltpu.SemaphoreType.DMA,
            pltpu.SemaphoreType.DMA,
        ],
    )
    def k(hs_h, row_h, col_h, zeros_h, out_h, row_i, col_i, buf_v, acc_sh,
          g0, g1):
        c = lax.axis_index("c")
        s = lax.axis_index("s")
        wb = (c * NS + s) * NB
        gsem = (g0, g1)

        pltpu.sync_copy(row_h.at[pl.ds(wb, NB)], row_i)
        pltpu.sync_copy(col_h.at[pl.ds(wb, NB)], col_i)

        def fire_g(p, b, j):
            pltpu.async_copy(hs_h.at[p].at[row_i.at[b]], buf_v.at[j], gsem[j])

        def wait_g(p, b, j):
            pltpu.make_async_copy(hs_h.at[p].at[row_i.at[b]], buf_v.at[j],
                                  gsem[j]).wait()

        def scat(b, j):
            pltpu.sync_copy(buf_v.at[j], acc_sh.at[col_i.at[b]], add=True)

        for p in range(P):
            pltpu.sync_copy(zeros_h, acc_sh.at[pl.ds(s * RPW, RPW)])
            plsc.subcore_barrier()

            fire_g(p, 0, 0)
            fire_g(p, 1, 1)

            @pl.loop(0, NB // 2 - 1)
            def _(k2):
                b0 = k2 * 2
                wait_g(p, b0, 0)
                scat(b0, 0)
                fire_g(p, b0 + 2, 0)
                wait_g(p, b0 + 1, 1)
                scat(b0 + 1, 1)
                fire_g(p, b0 + 3, 1)

            wait_g(p, NB - 2, 0)
            scat(NB - 2, 0)
            wait_g(p, NB - 1, 1)
            scat(NB - 1, 1)

            plsc.subcore_barrier()
            pltpu.sync_copy(acc_sh.at[pl.ds(s * RPW, RPW)],
                            out_h.at[c, p, pl.ds(s * RPW, RPW)])
            if p + 1 < P:
                plsc.subcore_barrier()

    return k(hs, row, col, zeros_c)


def _tc_dinv(degp):
    def body(deg_ref, dinv_ref):
        d = deg_ref[0] + deg_ref[1] + 1.0
        dinv_ref[...] = (1.0 / jnp.sqrt(d))[:, 0:16]

    return pl.pallas_call(
        body,
        grid=(NRB,),
        in_specs=[pl.BlockSpec((NC, RB, 128), lambda i: (0, i, 0))],
        out_specs=pl.BlockSpec((RB, 16), lambda i: (i, 0)),
        out_shape=jax.ShapeDtypeStruct((N, 16), jnp.float32),
    )(degp)


def _tc_matmul_scale(h_in, W, dinv16):
    K = h_in.shape[1]
    H = W.shape[1]
    P = H // 128

    def body(a_ref, w_ref, dinv_ref, out_ref):
        z = jnp.dot(a_ref[...], w_ref[...], preferred_element_type=jnp.float32)
        dv = dinv_ref[...][:, 0:1]
        for p in range(P):
            out_ref[p] = dv * z[:, p * 128:(p + 1) * 128]

    return pl.pallas_call(
        body,
        grid=(NRB,),
        in_specs=[
            pl.BlockSpec((RB, K), lambda i: (i, 0)),
            pl.BlockSpec((K, H), lambda i: (0, 0)),
            pl.BlockSpec((RB, 16), lambda i: (i, 0)),
        ],
        out_specs=pl.BlockSpec((P, RB, 128), lambda i: (0, i, 0)),
        out_shape=jax.ShapeDtypeStruct((P, N, 128), jnp.float32),
    )(h_in, W, dinv16)


def _tc_combine(partials, hs, dinv16, bias):
    P = hs.shape[0]
    H = P * 128

    def body(part_ref, hs_ref, dinv_ref, b_ref, h_ref, st_ref):
        i = pl.program_id(0)
        dv = dinv_ref[...][:, 0:1]
        agg = jnp.concatenate(
            [dv * (part_ref[0, p] + part_ref[1, p] + hs_ref[p])
             for p in range(P)], axis=1)
        h = jnp.maximum(agg + b_ref[...], 0.0)
        h_ref[...] = h
        s = jnp.sum(h)
        s2 = jnp.sum(h * h)
        vec = jnp.concatenate([jnp.full((1, 128), s, jnp.float32),
                               jnp.full((1, 128), s2, jnp.float32)], axis=1)

        @pl.when(i == 0)
        def _():
            st_ref[...] = jnp.zeros((1, 256), jnp.float32)

        st_ref[...] += vec

    return pl.pallas_call(
        body,
        grid=(NRB,),
        in_specs=[
            pl.BlockSpec((NC, P, RB, 128), lambda i: (0, 0, i, 0)),
            pl.BlockSpec((P, RB, 128), lambda i: (0, i, 0)),
            pl.BlockSpec((RB, 16), lambda i: (i, 0)),
            pl.BlockSpec((1, H), lambda i: (0, 0)),
        ],
        out_specs=[
            pl.BlockSpec((RB, H), lambda i: (i, 0)),
            pl.BlockSpec((1, 256), lambda i: (0, 0)),
        ],
        out_shape=[
            jax.ShapeDtypeStruct((N, H), jnp.float32),
            jax.ShapeDtypeStruct((1, 256), jnp.float32),
        ],
    )(partials, hs, dinv16, bias)


def _tc_ln(h, stats, lnw, lnb):
    nelem = float(N * HID)

    def body(h_ref, st_ref, w_ref, b_ref, out_ref):
        mu = st_ref[0, 0] / nelem
        ms = st_ref[0, 128] / nelem
        inv = 1.0 / (jnp.sqrt(jnp.maximum(ms - mu * mu, 0.0)) + EPS)
        out_ref[...] = (h_ref[...] - mu) * inv * w_ref[...] + b_ref[...]

    return pl.pallas_call(
        body,
        grid=(NRB,),
        in_specs=[
            pl.BlockSpec((RB, HID), lambda i: (i, 0)),
            pl.BlockSpec((1, 256), lambda i: (0, 0)),
            pl.BlockSpec((1, HID), lambda i: (0, 0)),
            pl.BlockSpec((1, HID), lambda i: (0, 0)),
        ],
        out_specs=pl.BlockSpec((RB, HID), lambda i: (i, 0)),
        out_shape=jax.ShapeDtypeStruct((N, HID), jnp.float32),
    )(h, stats, lnw, lnb)


def _tc_matmul3_scale(h_in, w3t, dinv16):
    def body(a_ref, w3_ref, dinv_ref, out_ref):
        t = jnp.dot(a_ref[...], w3_ref[...], preferred_element_type=jnp.float32)
        out_ref[...] = dinv_ref[...][:, 0:1] * t

    return pl.pallas_call(
        body,
        grid=(NRB,),
        in_specs=[
            pl.BlockSpec((RB, HID), lambda i: (i, 0)),
            pl.BlockSpec((HID, 128), lambda i: (0, 0)),
            pl.BlockSpec((RB, 16), lambda i: (i, 0)),
        ],
        out_specs=pl.BlockSpec((RB, 128), lambda i: (i, 0)),
        out_shape=jax.ShapeDtypeStruct((N, 128), jnp.float32),
    )(h_in, w3t, dinv16)


def _tc_final(p3, ts, dinv16, b3t, lnw3t, lnb3t):
    def body(p_ref, ts_ref, dinv_ref, b3_ref, w_ref, b_ref, out_ref):
        psum = (p_ref[0, 0] + p_ref[1, 0])[0:N, :]
        y = dinv_ref[...][:, 0:1] * (psum + ts_ref[...])
        h = jnp.maximum(y + b3_ref[...], 0.0)
        col = h[:, 0:1]
        mu = jnp.sum(col) / N
        ms = jnp.sum(col * col) / N
        inv = 1.0 / (jnp.sqrt(jnp.maximum(ms - mu * mu, 0.0)) + EPS)
        out_ref[...] = (h - mu) * inv * w_ref[...] + b_ref[...]

    return pl.pallas_call(
        body,
        in_specs=[
            pl.BlockSpec((NC, 1, ACC, 128), lambda: (0, 0, 0, 0)),
            pl.BlockSpec((N, 128), lambda: (0, 0)),
            pl.BlockSpec((N, 16), lambda: (0, 0)),
            pl.BlockSpec((1, 128), lambda: (0, 0)),
            pl.BlockSpec((1, 128), lambda: (0, 0)),
            pl.BlockSpec((1, 128), lambda: (0, 0)),
        ],
        out_specs=pl.BlockSpec((N, 128), lambda: (0, 0)),
        out_shape=jax.ShapeDtypeStruct((N, 128), jnp.float32),
    )(p3, ts, dinv16, b3t, lnw3t, lnb3t)


def kernel(x, edge_index, W1, b1, W2, b2, W3, b3,
           ln1_w, ln1_b, ln2_w, ln2_b, ln3_w, ln3_b):
    ei = edge_index.astype(jnp.int32)
    e = ei.shape[1]
    row = jnp.concatenate([ei[0], jnp.zeros((E_PAD - e,), jnp.int32)])
    col = jnp.concatenate([ei[1], jnp.full((E_PAD - e,), N, jnp.int32)])
    row = row.reshape(E_PAD // 128, 128)
    col = col.reshape(E_PAD // 128, 128)
    zeros128 = jnp.zeros((RPW, 128), jnp.float32)
    ones_b = jnp.ones((128, 128), jnp.float32)

    degp = _sc_degree(col, zeros128, ones_b)
    dinv16 = _tc_dinv(degp)

    hs0 = _tc_matmul_scale(x, W1, dinv16)
    p1 = _sc_aggregate(hs0, row, col, zeros128)
    h1, st1 = _tc_combine(p1, hs0, dinv16, b1.reshape(1, -1))
    h1n = _tc_ln(h1, st1, ln1_w.reshape(1, -1), ln1_b.reshape(1, -1))

    hs1 = _tc_matmul_scale(h1n, W2, dinv16)
    p2 = _sc_aggregate(hs1, row, col, zeros128)
    h2, st2 = _tc_combine(p2, hs1, dinv16, b2.reshape(1, -1))
    h2n = _tc_ln(h2, st2, ln2_w.reshape(1, -1), ln2_b.reshape(1, -1))

    ts = _tc_matmul3_scale(h2n, jnp.tile(W3, (1, 128)), dinv16)
    p3 = _sc_aggregate(ts.reshape(1, N, 128), row, col, zeros128)
    out128 = _tc_final(p3, ts, dinv16,
                       jnp.broadcast_to(b3.reshape(1, 1), (1, 128)),
                       jnp.broadcast_to(ln3_w.reshape(1, 1), (1, 128)),
                       jnp.broadcast_to(ln3_b.reshape(1, 1), (1, 128)))
    return out128[:, 0]
